# Initial kernel scaffold; baseline (speedup 1.0000x reference)
import jax
import jax.numpy as jnp
from jax.experimental import pallas as pl


def kernel(variances, ins_labels):
    raise NotImplementedError("write your pallas kernel here")


# speedup vs baseline: 6.4815x; 6.4815x over previous
import functools

import jax
import jax.numpy as jnp
from jax import lax
from jax.experimental import pallas as pl
from jax.experimental.pallas import tpu as pltpu
from jax.experimental.pallas import tpu_sc as plsc

N = 320000
D = 128
K = 1024
NC = 2
NS = 16
NW = NC * NS
LANES = 16
ROWS_W = N // NW
CH = 400
NCHUNK = ROWS_W // CH
AUXW = D

_mesh = plsc.VectorSubcoreMesh(core_axis_name="c", subcore_axis_name="s")


@functools.partial(
    pl.kernel,
    out_type=[
        jax.ShapeDtypeStruct((NC, K, D), jnp.float32),
        jax.ShapeDtypeStruct((NC, K, AUXW), jnp.float32),
    ],
    mesh=_mesh,
    scratch_types=[
        pltpu.VMEM((CH, D), jnp.float32),
        pltpu.VMEM((CH, AUXW), jnp.float32),
        pltpu.VMEM((CH,), jnp.int32),
        pltpu.VMEM_SHARED((K, D), jnp.float32),
        pltpu.VMEM_SHARED((K, AUXW), jnp.float32),
    ],
)
def _accum_kernel(var_hbm, lab_hbm, sums_out, aux_out,
                  xbuf, zbuf, idxbuf, ssums, saux):
    cid = lax.axis_index("c")
    sid = lax.axis_index("s")
    wid = cid * NS + sid
    zeros16 = jnp.zeros((LANES,), jnp.float32)

    def _zrow(r, _):
        for j in range(D // LANES):
            xbuf[r, pl.ds(j * LANES, LANES)] = zeros16
        for j in range(AUXW // LANES):
            zbuf[r, pl.ds(j * LANES, LANES)] = zeros16
        return 0
    lax.fori_loop(0, CH, _zrow, 0)

    @pl.when(sid == 0)
    def _():
        pltpu.sync_copy(xbuf, ssums.at[pl.ds(0, CH)])
        pltpu.sync_copy(xbuf, ssums.at[pl.ds(CH, CH)])
        pltpu.sync_copy(xbuf.at[pl.ds(0, K - 2 * CH)],
                        ssums.at[pl.ds(2 * CH, K - 2 * CH)])
        pltpu.sync_copy(zbuf, saux.at[pl.ds(0, CH)])
        pltpu.sync_copy(zbuf, saux.at[pl.ds(CH, CH)])
        pltpu.sync_copy(zbuf.at[pl.ds(0, K - 2 * CH)],
                        saux.at[pl.ds(2 * CH, K - 2 * CH)])
    plsc.subcore_barrier()

    lanes = jnp.arange(LANES, dtype=jnp.int32)
    onehot = jnp.where(lanes == 0, 1.0, 0.0).astype(jnp.float32)

    def _crow(r, _):
        zbuf[r, pl.ds(LANES, LANES)] = onehot
        return 0
    lax.fori_loop(0, CH, _crow, 0)

    def _chunk(i, _):
        base = pl.multiple_of(wid * ROWS_W + i * CH, 8)
        pltpu.sync_copy(var_hbm.at[pl.ds(base, CH)], xbuf)
        pltpu.sync_copy(lab_hbm.at[pl.ds(base, CH)], idxbuf)

        def _row(r, _):
            acc = zeros16
            for j in range(D // LANES):
                v = xbuf[r, pl.ds(j * LANES, LANES)]
                acc = acc + v * v
            zbuf[r, pl.ds(0, LANES)] = acc
            return 0
        lax.fori_loop(0, CH, _row, 0)

        pltpu.sync_copy(xbuf, ssums.at[idxbuf], add=True)
        pltpu.sync_copy(zbuf, saux.at[idxbuf], add=True)
        return 0
    lax.fori_loop(0, NCHUNK, _chunk, 0)

    plsc.subcore_barrier()

    @pl.when(sid == 0)
    def _():
        pltpu.sync_copy(ssums, sums_out.at[cid])
        pltpu.sync_copy(saux, aux_out.at[cid])


SCH = 128


@functools.partial(
    pl.kernel,
    out_type=jax.ShapeDtypeStruct((LANES,), jnp.float32),
    mesh=_mesh,
    scratch_types=[
        pltpu.VMEM((SCH, D), jnp.float32),
        pltpu.VMEM((SCH, D), jnp.float32),
        pltpu.VMEM((SCH, AUXW), jnp.float32),
        pltpu.VMEM((SCH, AUXW), jnp.float32),
        pltpu.VMEM((LANES,), jnp.float32),
    ],
)
def _loss_kernel(sums_p, aux_p, out_hbm, s0, s1, a0, a1, outbuf):
    cid = lax.axis_index("c")
    sid = lax.axis_index("s")

    @pl.when((cid == 0) & (sid == 0))
    def _():
        lanes = jnp.arange(LANES, dtype=jnp.int32)
        zeros16 = jnp.zeros((LANES,), jnp.float32)

        def _chunk(ci, carry):
            loss_s, nuniq_s = carry
            off = ci * SCH
            pltpu.sync_copy(sums_p.at[0, pl.ds(off, SCH)], s0)
            pltpu.sync_copy(sums_p.at[1, pl.ds(off, SCH)], s1)
            pltpu.sync_copy(aux_p.at[0, pl.ds(off, SCH)], a0)
            pltpu.sync_copy(aux_p.at[1, pl.ds(off, SCH)], a1)

            def _seg(r, carry2):
                loss_v, nuniq_s = carry2
                sacc = zeros16
                for j in range(D // LANES):
                    v = (s0[r, pl.ds(j * LANES, LANES)] +
                         s1[r, pl.ds(j * LANES, LANES)])
                    sacc = sacc + v * v
                qv = a0[r, pl.ds(0, LANES)] + a1[r, pl.ds(0, LANES)]
                cv = (a0[r, pl.ds(LANES, LANES)] +
                      a1[r, pl.ds(LANES, LANES)])
                c = cv[0]

                segid = off + r
                present = c > 0.0
                valid = present & (segid != 0)
                safec = jnp.where(present, c, 1.0)
                contrib = jnp.where(
                    valid, (qv - sacc / safec) / (safec * float(D)),
                    zeros16)
                nuniq_s = nuniq_s + jnp.where(present, 1.0, 0.0)
                return loss_v + contrib, nuniq_s

            return lax.fori_loop(0, SCH, _seg, (loss_s, nuniq_s))

        loss_v, nuniq = lax.fori_loop(0, K // SCH, _chunk,
                                      (zeros16, 0.0))
        loss = loss_v[0]
        for i in range(1, LANES):
            loss = loss + loss_v[i]
        loss = jnp.where(nuniq == 1.0, 0.0, loss)
        outbuf[pl.ds(0, LANES)] = jnp.where(lanes == 0, loss, 0.0)
        pltpu.sync_copy(outbuf, out_hbm)


def kernel(variances, ins_labels):
    sums_p, aux_p = _accum_kernel(variances, ins_labels.astype(jnp.int32))
    out = _loss_kernel(sums_p, aux_p)
    return out[0]


# speedup vs baseline: 7.4177x; 1.1444x over previous
import functools

import jax
import jax.numpy as jnp
from jax import lax
from jax.experimental import pallas as pl
from jax.experimental.pallas import tpu as pltpu
from jax.experimental.pallas import tpu_sc as plsc

N = 320000
D = 128
K = 1024
NC = 2
NS = 16
NW = NC * NS
LANES = 16
ROWS_W = N // NW
CH = 200
NCHUNK = ROWS_W // CH
AUXW = D

_mesh = plsc.VectorSubcoreMesh(core_axis_name="c", subcore_axis_name="s")


@functools.partial(
    pl.kernel,
    out_type=[
        jax.ShapeDtypeStruct((NC, K, D), jnp.float32),
        jax.ShapeDtypeStruct((NC, K, AUXW), jnp.float32),
    ],
    mesh=_mesh,
    scratch_types=[
        pltpu.VMEM((CH, D), jnp.float32),
        pltpu.VMEM((CH, D), jnp.float32),
        pltpu.VMEM((CH, AUXW), jnp.float32),
        pltpu.VMEM((CH, AUXW), jnp.float32),
        pltpu.VMEM((CH,), jnp.int32),
        pltpu.VMEM((CH,), jnp.int32),
        pltpu.VMEM_SHARED((K, D), jnp.float32),
        pltpu.VMEM_SHARED((K, AUXW), jnp.float32),
        pltpu.SemaphoreType.DMA,
        pltpu.SemaphoreType.DMA,
        pltpu.SemaphoreType.DMA,
        pltpu.SemaphoreType.DMA,
        pltpu.SemaphoreType.DMA,
        pltpu.SemaphoreType.DMA,
        pltpu.SemaphoreType.DMA,
        pltpu.SemaphoreType.DMA,
    ],
)
def _accum_kernel(var_hbm, lab_hbm, sums_out, aux_out,
                  x0, x1, z0, z1, i0, i1, ssums, saux,
                  inx0, inx1, inl0, inl1, scx0, scx1, scz0, scz1):
    cid = lax.axis_index("c")
    sid = lax.axis_index("s")
    wid = cid * NS + sid
    zeros16 = jnp.zeros((LANES,), jnp.float32)
    xb = (x0, x1)
    zb = (z0, z1)
    ib = (i0, i1)
    inx = (inx0, inx1)
    inl = (inl0, inl1)
    scx = (scx0, scx1)
    scz = (scz0, scz1)

    def _zrow(r, _):
        for j in range(D // LANES):
            x0[r, pl.ds(j * LANES, LANES)] = zeros16
        for j in range(AUXW // LANES):
            z0[r, pl.ds(j * LANES, LANES)] = zeros16
        return 0
    lax.fori_loop(0, CH, _zrow, 0)

    @pl.when(sid == 0)
    def _():
        for part in range(K // CH):
            pltpu.sync_copy(x0, ssums.at[pl.ds(part * CH, CH)])
            pltpu.sync_copy(z0, saux.at[pl.ds(part * CH, CH)])
        rem = K - (K // CH) * CH
        if rem:
            pltpu.sync_copy(x0.at[pl.ds(0, rem)],
                            ssums.at[pl.ds(K - rem, rem)])
            pltpu.sync_copy(z0.at[pl.ds(0, rem)],
                            saux.at[pl.ds(K - rem, rem)])
    plsc.subcore_barrier()

    lanes = jnp.arange(LANES, dtype=jnp.int32)
    onehot = jnp.where(lanes == 0, 1.0, 0.0).astype(jnp.float32)

    def _crow(r, _):
        z0[r, pl.ds(LANES, LANES)] = onehot
        z1[r, pl.ds(LANES, LANES)] = onehot
        return 0
    lax.fori_loop(0, CH, _crow, 0)

    row0 = wid * ROWS_W
    pltpu.async_copy(var_hbm.at[pl.ds(row0, CH)], x0, inx[0])
    pltpu.async_copy(lab_hbm.at[pl.ds(row0, CH)], i0, inl[0])

    def _pair(p, _):
        base_p = pl.multiple_of(row0 + p * 2 * CH, 8)
        for b in (0, 1):
            pltpu.make_async_copy(
                var_hbm.at[pl.ds(base_p + b * CH, CH)], xb[b],
                inx[b]).wait()
            pltpu.make_async_copy(
                lab_hbm.at[pl.ds(base_p + b * CH, CH)], ib[b],
                inl[b]).wait()

            def _row(r, _):
                acc = zeros16
                for j in range(D // LANES):
                    v = xb[b][r, pl.ds(j * LANES, LANES)]
                    acc = acc + v * v
                zb[b][r, pl.ds(0, LANES)] = acc
                return 0
            lax.fori_loop(0, CH, _row, 0)

            pltpu.async_copy(xb[b], ssums.at[ib[b]], scx[b], add=True)
            pltpu.async_copy(zb[b], saux.at[ib[b]], scz[b], add=True)

            ob = 1 - b
            nbase = pl.multiple_of(base_p + (b + 1) * CH, 8)

            def _prefetch(first):
                if not first:
                    pltpu.make_async_copy(
                        xb[ob], ssums.at[ib[ob]], scx[ob]).wait()
                    pltpu.make_async_copy(
                        zb[ob], saux.at[ib[ob]], scz[ob]).wait()
                pltpu.async_copy(var_hbm.at[pl.ds(nbase, CH)],
                                 xb[ob], inx[ob])
                pltpu.async_copy(lab_hbm.at[pl.ds(nbase, CH)],
                                 ib[ob], inl[ob])

            if b == 0:
                @pl.when(p > 0)
                def _():
                    _prefetch(False)

                @pl.when(p == 0)
                def _():
                    _prefetch(True)
            else:
                @pl.when(p + 1 < NCHUNK // 2)
                def _():
                    _prefetch(False)
        return 0
    lax.fori_loop(0, NCHUNK // 2, _pair, 0)

    for b in (0, 1):
        pltpu.make_async_copy(xb[b], ssums.at[ib[b]], scx[b]).wait()
        pltpu.make_async_copy(zb[b], saux.at[ib[b]], scz[b]).wait()

    plsc.subcore_barrier()

    @pl.when(sid == 0)
    def _():
        pltpu.sync_copy(ssums, sums_out.at[cid])
        pltpu.sync_copy(saux, aux_out.at[cid])


SCH = 128


@functools.partial(
    pl.kernel,
    out_type=jax.ShapeDtypeStruct((LANES,), jnp.float32),
    mesh=_mesh,
    scratch_types=[
        pltpu.VMEM((SCH, D), jnp.float32),
        pltpu.VMEM((SCH, D), jnp.float32),
        pltpu.VMEM((SCH, AUXW), jnp.float32),
        pltpu.VMEM((SCH, AUXW), jnp.float32),
        pltpu.VMEM((LANES,), jnp.float32),
    ],
)
def _loss_kernel(sums_p, aux_p, out_hbm, s0, s1, a0, a1, outbuf):
    cid = lax.axis_index("c")
    sid = lax.axis_index("s")

    @pl.when((cid == 0) & (sid == 0))
    def _():
        lanes = jnp.arange(LANES, dtype=jnp.int32)
        zeros16 = jnp.zeros((LANES,), jnp.float32)

        def _chunk(ci, carry):
            loss_s, nuniq_s = carry
            off = ci * SCH
            pltpu.sync_copy(sums_p.at[0, pl.ds(off, SCH)], s0)
            pltpu.sync_copy(sums_p.at[1, pl.ds(off, SCH)], s1)
            pltpu.sync_copy(aux_p.at[0, pl.ds(off, SCH)], a0)
            pltpu.sync_copy(aux_p.at[1, pl.ds(off, SCH)], a1)

            def _seg(r, carry2):
                loss_v, nuniq_s = carry2
                sacc = zeros16
                for j in range(D // LANES):
                    v = (s0[r, pl.ds(j * LANES, LANES)] +
                         s1[r, pl.ds(j * LANES, LANES)])
                    sacc = sacc + v * v
                qv = a0[r, pl.ds(0, LANES)] + a1[r, pl.ds(0, LANES)]
                cv = (a0[r, pl.ds(LANES, LANES)] +
                      a1[r, pl.ds(LANES, LANES)])
                c = cv[0]

                segid = off + r
                present = c > 0.0
                valid = present & (segid != 0)
                safec = jnp.where(present, c, 1.0)
                contrib = jnp.where(
                    valid, (qv - sacc / safec) / (safec * float(D)),
                    zeros16)
                nuniq_s = nuniq_s + jnp.where(present, 1.0, 0.0)
                return loss_v + contrib, nuniq_s

            return lax.fori_loop(0, SCH, _seg, (loss_s, nuniq_s))

        loss_v, nuniq = lax.fori_loop(0, K // SCH, _chunk,
                                      (zeros16, 0.0))
        loss = loss_v[0]
        for i in range(1, LANES):
            loss = loss + loss_v[i]
        loss = jnp.where(nuniq == 1.0, 0.0, loss)
        outbuf[pl.ds(0, LANES)] = jnp.where(lanes == 0, loss, 0.0)
        pltpu.sync_copy(outbuf, out_hbm)


def kernel(variances, ins_labels):
    sums_p, aux_p = _accum_kernel(variances, ins_labels.astype(jnp.int32))
    out = _loss_kernel(sums_p, aux_p)
    return out[0]


# speedup vs baseline: 8.5344x; 1.1505x over previous
import functools

import jax
import jax.numpy as jnp
from jax import lax
from jax.experimental import pallas as pl
from jax.experimental.pallas import tpu as pltpu
from jax.experimental.pallas import tpu_sc as plsc

N = 320000
D = 128
K = 1024
NC = 2
NS = 16
NW = NC * NS
LANES = 16
ROWS_W = N // NW
CH = 200
NCHUNK = ROWS_W // CH
AUXW = D

_mesh = plsc.VectorSubcoreMesh(core_axis_name="c", subcore_axis_name="s")


@functools.partial(
    pl.kernel,
    out_type=[
        jax.ShapeDtypeStruct((NC, K, D), jnp.float32),
        jax.ShapeDtypeStruct((NC, K, AUXW), jnp.float32),
    ],
    mesh=_mesh,
    scratch_types=[
        pltpu.VMEM((CH, D), jnp.float32),
        pltpu.VMEM((CH, D), jnp.float32),
        pltpu.VMEM((CH, AUXW), jnp.float32),
        pltpu.VMEM((CH, AUXW), jnp.float32),
        pltpu.VMEM((CH,), jnp.int32),
        pltpu.VMEM((CH,), jnp.int32),
        pltpu.VMEM_SHARED((K, D), jnp.float32),
        pltpu.VMEM_SHARED((K, AUXW), jnp.float32),
        pltpu.SemaphoreType.DMA,
        pltpu.SemaphoreType.DMA,
        pltpu.SemaphoreType.DMA,
        pltpu.SemaphoreType.DMA,
        pltpu.SemaphoreType.DMA,
        pltpu.SemaphoreType.DMA,
        pltpu.SemaphoreType.DMA,
        pltpu.SemaphoreType.DMA,
    ],
)
def _accum_kernel(var_hbm, lab_hbm, sums_out, aux_out,
                  x0, x1, z0, z1, i0, i1, ssums, saux,
                  inx0, inx1, inl0, inl1, scx0, scx1, scz0, scz1):
    cid = lax.axis_index("c")
    sid = lax.axis_index("s")
    wid = cid * NS + sid
    zeros16 = jnp.zeros((LANES,), jnp.float32)
    xb = (x0, x1)
    zb = (z0, z1)
    ib = (i0, i1)
    inx = (inx0, inx1)
    inl = (inl0, inl1)
    scx = (scx0, scx1)
    scz = (scz0, scz1)

    def _zrow(r, _):
        for j in range(D // LANES):
            x0[r, pl.ds(j * LANES, LANES)] = zeros16
        for j in range(AUXW // LANES):
            z0[r, pl.ds(j * LANES, LANES)] = zeros16
        return 0
    lax.fori_loop(0, CH, _zrow, 0)

    @pl.when(sid == 0)
    def _():
        for part in range(K // CH):
            pltpu.sync_copy(x0, ssums.at[pl.ds(part * CH, CH)])
            pltpu.sync_copy(z0, saux.at[pl.ds(part * CH, CH)])
        rem = K - (K // CH) * CH
        if rem:
            pltpu.sync_copy(x0.at[pl.ds(0, rem)],
                            ssums.at[pl.ds(K - rem, rem)])
            pltpu.sync_copy(z0.at[pl.ds(0, rem)],
                            saux.at[pl.ds(K - rem, rem)])
    plsc.subcore_barrier()

    lanes = jnp.arange(LANES, dtype=jnp.int32)
    onehot = jnp.where(lanes == 0, 1.0, 0.0).astype(jnp.float32)

    def _crow(r, _):
        z0[r, pl.ds(LANES, LANES)] = onehot
        z1[r, pl.ds(LANES, LANES)] = onehot
        return 0
    lax.fori_loop(0, CH, _crow, 0)

    row0 = wid * ROWS_W
    pltpu.async_copy(var_hbm.at[pl.ds(row0, CH)], x0, inx[0])
    pltpu.async_copy(lab_hbm.at[pl.ds(row0, CH)], i0, inl[0])

    def _pair(p, _):
        base_p = pl.multiple_of(row0 + p * 2 * CH, 8)
        for b in (0, 1):
            pltpu.make_async_copy(
                var_hbm.at[pl.ds(base_p + b * CH, CH)], xb[b],
                inx[b]).wait()
            pltpu.make_async_copy(
                lab_hbm.at[pl.ds(base_p + b * CH, CH)], ib[b],
                inl[b]).wait()

            def _row(r, _):
                acc = zeros16
                for j in range(D // LANES):
                    v = xb[b][r, pl.ds(j * LANES, LANES)]
                    acc = acc + v * v
                zb[b][r, pl.ds(0, LANES)] = acc
                return 0
            lax.fori_loop(0, CH, _row, 0)

            pltpu.async_copy(xb[b], ssums.at[ib[b]], scx[b], add=True)
            pltpu.async_copy(zb[b], saux.at[ib[b]], scz[b], add=True)

            ob = 1 - b
            nbase = pl.multiple_of(base_p + (b + 1) * CH, 8)

            def _prefetch(first):
                if not first:
                    pltpu.make_async_copy(
                        xb[ob], ssums.at[ib[ob]], scx[ob]).wait()
                    pltpu.make_async_copy(
                        zb[ob], saux.at[ib[ob]], scz[ob]).wait()
                pltpu.async_copy(var_hbm.at[pl.ds(nbase, CH)],
                                 xb[ob], inx[ob])
                pltpu.async_copy(lab_hbm.at[pl.ds(nbase, CH)],
                                 ib[ob], inl[ob])

            if b == 0:
                @pl.when(p > 0)
                def _():
                    _prefetch(False)

                @pl.when(p == 0)
                def _():
                    _prefetch(True)
            else:
                @pl.when(p + 1 < NCHUNK // 2)
                def _():
                    _prefetch(False)
        return 0
    lax.fori_loop(0, NCHUNK // 2, _pair, 0)

    for b in (0, 1):
        pltpu.make_async_copy(xb[b], ssums.at[ib[b]], scx[b]).wait()
        pltpu.make_async_copy(zb[b], saux.at[ib[b]], scz[b]).wait()

    plsc.subcore_barrier()

    @pl.when(sid == 0)
    def _():
        pltpu.sync_copy(ssums, sums_out.at[cid])
        pltpu.sync_copy(saux, aux_out.at[cid])


SCH = K // NS


@functools.partial(
    pl.kernel,
    out_type=jax.ShapeDtypeStruct((LANES,), jnp.float32),
    mesh=_mesh,
    scratch_types=[
        pltpu.VMEM((SCH, D), jnp.float32),
        pltpu.VMEM((SCH, D), jnp.float32),
        pltpu.VMEM((SCH, AUXW), jnp.float32),
        pltpu.VMEM((SCH, AUXW), jnp.float32),
        pltpu.VMEM((D,), jnp.float32),
        pltpu.VMEM((NS, D), jnp.float32),
        pltpu.VMEM((LANES,), jnp.float32),
        pltpu.VMEM_SHARED((NS, D), jnp.float32),
    ],
)
def _loss_kernel(sums_p, aux_p, out_hbm, s0, s1, a0, a1,
                 stg, fold, outbuf, sstage):
    cid = lax.axis_index("c")
    sid = lax.axis_index("s")

    @pl.when(cid == 0)
    def _():
        lanes = jnp.arange(LANES, dtype=jnp.int32)
        zeros16 = jnp.zeros((LANES,), jnp.float32)
        off = sid * SCH
        pltpu.sync_copy(sums_p.at[0, pl.ds(off, SCH)], s0)
        pltpu.sync_copy(sums_p.at[1, pl.ds(off, SCH)], s1)
        pltpu.sync_copy(aux_p.at[0, pl.ds(off, SCH)], a0)
        pltpu.sync_copy(aux_p.at[1, pl.ds(off, SCH)], a1)

        def _seg(r, carry):
            loss_v, nuniq_s = carry
            sacc = zeros16
            for j in range(D // LANES):
                v = (s0[r, pl.ds(j * LANES, LANES)] +
                     s1[r, pl.ds(j * LANES, LANES)])
                sacc = sacc + v * v
            qv = a0[r, pl.ds(0, LANES)] + a1[r, pl.ds(0, LANES)]
            cv = (a0[r, pl.ds(LANES, LANES)] +
                  a1[r, pl.ds(LANES, LANES)])
            c = cv[0]

            segid = off + r
            present = c > 0.0
            valid = present & (segid != 0)
            safec = jnp.where(present, c, 1.0)
            contrib = jnp.where(
                valid, (qv - sacc / safec) / (safec * float(D)),
                zeros16)
            nuniq_s = nuniq_s + jnp.where(present, 1.0, 0.0)
            return loss_v + contrib, nuniq_s

        loss_v, nuniq = lax.fori_loop(0, SCH, _seg, (zeros16, 0.0))

        stg[pl.ds(0, LANES)] = loss_v
        stg[pl.ds(LANES, LANES)] = jnp.where(lanes == 0, nuniq, 0.0)
        for j in range(2, D // LANES):
            stg[pl.ds(j * LANES, LANES)] = zeros16
        pltpu.sync_copy(stg, sstage.at[sid])
        plsc.subcore_barrier()

        @pl.when(sid == 0)
        def _():
            pltpu.sync_copy(sstage, fold)
            lv = zeros16
            nv = zeros16
            for t in range(NS):
                lv = lv + fold[t, pl.ds(0, LANES)]
                nv = nv + fold[t, pl.ds(LANES, LANES)]
            loss = lv[0]
            for i in range(1, LANES):
                loss = loss + lv[i]
            nuniq_tot = nv[0]
            loss = jnp.where(nuniq_tot == 1.0, 0.0, loss)
            outbuf[pl.ds(0, LANES)] = jnp.where(lanes == 0, loss, 0.0)
            pltpu.sync_copy(outbuf, out_hbm)


def kernel(variances, ins_labels):
    sums_p, aux_p = _accum_kernel(variances, ins_labels.astype(jnp.int32))
    out = _loss_kernel(sums_p, aux_p)
    return out[0]


# speedup vs baseline: 9.4564x; 1.1080x over previous
import functools

import jax
import jax.numpy as jnp
from jax import lax
from jax.experimental import pallas as pl
from jax.experimental.pallas import tpu as pltpu
from jax.experimental.pallas import tpu_sc as plsc

N = 320000
D = 128
K = 1024
NC = 2
NS = 16
NW = NC * NS
LANES = 16
ROWS_W = N // NW
CH = 200
NCHUNK = ROWS_W // CH
AUXW = 2 * LANES

_mesh = plsc.VectorSubcoreMesh(core_axis_name="c", subcore_axis_name="s")


@functools.partial(
    pl.kernel,
    out_type=[
        jax.ShapeDtypeStruct((NC, K, D), jnp.float32),
        jax.ShapeDtypeStruct((NC, K, AUXW), jnp.float32),
    ],
    mesh=_mesh,
    compiler_params=pltpu.CompilerParams(use_tc_tiling_on_sc=False),
    scratch_types=[
        pltpu.VMEM((CH, D), jnp.float32),
        pltpu.VMEM((CH, D), jnp.float32),
        pltpu.VMEM((CH, AUXW), jnp.float32),
        pltpu.VMEM((CH, AUXW), jnp.float32),
        pltpu.VMEM((CH,), jnp.int32),
        pltpu.VMEM((CH,), jnp.int32),
        pltpu.VMEM_SHARED((K, D), jnp.float32),
        pltpu.VMEM_SHARED((K, AUXW), jnp.float32),
        pltpu.SemaphoreType.DMA,
        pltpu.SemaphoreType.DMA,
        pltpu.SemaphoreType.DMA,
        pltpu.SemaphoreType.DMA,
        pltpu.SemaphoreType.DMA,
        pltpu.SemaphoreType.DMA,
        pltpu.SemaphoreType.DMA,
        pltpu.SemaphoreType.DMA,
    ],
)
def _accum_kernel(var_hbm, lab_hbm, sums_out, aux_out,
                  x0, x1, z0, z1, i0, i1, ssums, saux,
                  inx0, inx1, inl0, inl1, scx0, scx1, scz0, scz1):
    cid = lax.axis_index("c")
    sid = lax.axis_index("s")
    wid = cid * NS + sid
    zeros16 = jnp.zeros((LANES,), jnp.float32)
    xb = (x0, x1)
    zb = (z0, z1)
    ib = (i0, i1)
    inx = (inx0, inx1)
    inl = (inl0, inl1)
    scx = (scx0, scx1)
    scz = (scz0, scz1)

    def _zrow(r, _):
        for j in range(D // LANES):
            x0[r, pl.ds(j * LANES, LANES)] = zeros16
        for j in range(AUXW // LANES):
            z0[r, pl.ds(j * LANES, LANES)] = zeros16
        return 0
    lax.fori_loop(0, CH, _zrow, 0)

    @pl.when(sid == 0)
    def _():
        for part in range(K // CH):
            pltpu.sync_copy(x0, ssums.at[pl.ds(part * CH, CH)])
            pltpu.sync_copy(z0, saux.at[pl.ds(part * CH, CH)])
        rem = K - (K // CH) * CH
        if rem:
            pltpu.sync_copy(x0.at[pl.ds(0, rem)],
                            ssums.at[pl.ds(K - rem, rem)])
            pltpu.sync_copy(z0.at[pl.ds(0, rem)],
                            saux.at[pl.ds(K - rem, rem)])
    plsc.subcore_barrier()

    lanes = jnp.arange(LANES, dtype=jnp.int32)
    onehot = jnp.where(lanes == 0, 1.0, 0.0).astype(jnp.float32)

    def _crow(r, _):
        z0[r, pl.ds(LANES, LANES)] = onehot
        z1[r, pl.ds(LANES, LANES)] = onehot
        return 0
    lax.fori_loop(0, CH, _crow, 0)

    row0 = wid * ROWS_W
    pltpu.async_copy(var_hbm.at[pl.ds(row0, CH)], x0, inx[0])
    pltpu.async_copy(lab_hbm.at[pl.ds(row0, CH)], i0, inl[0])

    def _pair(p, _):
        base_p = pl.multiple_of(row0 + p * 2 * CH, 8)
        for b in (0, 1):
            pltpu.make_async_copy(
                var_hbm.at[pl.ds(base_p + b * CH, CH)], xb[b],
                inx[b]).wait()
            pltpu.make_async_copy(
                lab_hbm.at[pl.ds(base_p + b * CH, CH)], ib[b],
                inl[b]).wait()

            def _row(r, _):
                acc = zeros16
                for j in range(D // LANES):
                    v = xb[b][r, pl.ds(j * LANES, LANES)]
                    acc = acc + v * v
                zb[b][r, pl.ds(0, LANES)] = acc
                return 0
            lax.fori_loop(0, CH, _row, 0)

            pltpu.async_copy(xb[b], ssums.at[ib[b]], scx[b], add=True)
            pltpu.async_copy(zb[b], saux.at[ib[b]], scz[b], add=True)

            ob = 1 - b
            nbase = pl.multiple_of(base_p + (b + 1) * CH, 8)

            def _prefetch(first):
                if not first:
                    pltpu.make_async_copy(
                        xb[ob], ssums.at[ib[ob]], scx[ob]).wait()
                    pltpu.make_async_copy(
                        zb[ob], saux.at[ib[ob]], scz[ob]).wait()
                pltpu.async_copy(var_hbm.at[pl.ds(nbase, CH)],
                                 xb[ob], inx[ob])
                pltpu.async_copy(lab_hbm.at[pl.ds(nbase, CH)],
                                 ib[ob], inl[ob])

            if b == 0:
                @pl.when(p > 0)
                def _():
                    _prefetch(False)

                @pl.when(p == 0)
                def _():
                    _prefetch(True)
            else:
                @pl.when(p + 1 < NCHUNK // 2)
                def _():
                    _prefetch(False)
        return 0
    lax.fori_loop(0, NCHUNK // 2, _pair, 0)

    for b in (0, 1):
        pltpu.make_async_copy(xb[b], ssums.at[ib[b]], scx[b]).wait()
        pltpu.make_async_copy(zb[b], saux.at[ib[b]], scz[b]).wait()

    plsc.subcore_barrier()

    @pl.when(sid == 0)
    def _():
        pltpu.sync_copy(ssums, sums_out.at[cid])
        pltpu.sync_copy(saux, aux_out.at[cid])


SCH = K // NS


@functools.partial(
    pl.kernel,
    out_type=jax.ShapeDtypeStruct((LANES,), jnp.float32),
    mesh=_mesh,
    compiler_params=pltpu.CompilerParams(use_tc_tiling_on_sc=False),
    scratch_types=[
        pltpu.VMEM((SCH, D), jnp.float32),
        pltpu.VMEM((SCH, D), jnp.float32),
        pltpu.VMEM((SCH, AUXW), jnp.float32),
        pltpu.VMEM((SCH, AUXW), jnp.float32),
        pltpu.VMEM((D,), jnp.float32),
        pltpu.VMEM((NS, D), jnp.float32),
        pltpu.VMEM((LANES,), jnp.float32),
        pltpu.VMEM_SHARED((NS, D), jnp.float32),
    ],
)
def _loss_kernel(sums_p, aux_p, out_hbm, s0, s1, a0, a1,
                 stg, fold, outbuf, sstage):
    cid = lax.axis_index("c")
    sid = lax.axis_index("s")

    @pl.when(cid == 0)
    def _():
        lanes = jnp.arange(LANES, dtype=jnp.int32)
        zeros16 = jnp.zeros((LANES,), jnp.float32)
        off = sid * SCH
        pltpu.sync_copy(sums_p.at[0, pl.ds(off, SCH)], s0)
        pltpu.sync_copy(sums_p.at[1, pl.ds(off, SCH)], s1)
        pltpu.sync_copy(aux_p.at[0, pl.ds(off, SCH)], a0)
        pltpu.sync_copy(aux_p.at[1, pl.ds(off, SCH)], a1)

        def _seg(r, carry):
            loss_v, nuniq_s = carry
            sacc = zeros16
            for j in range(D // LANES):
                v = (s0[r, pl.ds(j * LANES, LANES)] +
                     s1[r, pl.ds(j * LANES, LANES)])
                sacc = sacc + v * v
            qv = a0[r, pl.ds(0, LANES)] + a1[r, pl.ds(0, LANES)]
            cv = (a0[r, pl.ds(LANES, LANES)] +
                  a1[r, pl.ds(LANES, LANES)])
            c = cv[0]

            segid = off + r
            present = c > 0.0
            valid = present & (segid != 0)
            safec = jnp.where(present, c, 1.0)
            contrib = jnp.where(
                valid, (qv - sacc / safec) / (safec * float(D)),
                zeros16)
            nuniq_s = nuniq_s + jnp.where(present, 1.0, 0.0)
            return loss_v + contrib, nuniq_s

        loss_v, nuniq = lax.fori_loop(0, SCH, _seg, (zeros16, 0.0))

        stg[pl.ds(0, LANES)] = loss_v
        stg[pl.ds(LANES, LANES)] = jnp.where(lanes == 0, nuniq, 0.0)
        for j in range(2, D // LANES):
            stg[pl.ds(j * LANES, LANES)] = zeros16
        pltpu.sync_copy(stg, sstage.at[sid])
        plsc.subcore_barrier()

        @pl.when(sid == 0)
        def _():
            pltpu.sync_copy(sstage, fold)
            lv = zeros16
            nv = zeros16
            for t in range(NS):
                lv = lv + fold[t, pl.ds(0, LANES)]
                nv = nv + fold[t, pl.ds(LANES, LANES)]
            loss = lv[0]
            for i in range(1, LANES):
                loss = loss + lv[i]
            nuniq_tot = nv[0]
            loss = jnp.where(nuniq_tot == 1.0, 0.0, loss)
            outbuf[pl.ds(0, LANES)] = jnp.where(lanes == 0, loss, 0.0)
            pltpu.sync_copy(outbuf, out_hbm)


def kernel(variances, ins_labels):
    sums_p, aux_p = _accum_kernel(variances, ins_labels.astype(jnp.int32))
    out = _loss_kernel(sums_p, aux_p)
    return out[0]


# speedup vs baseline: 12.7331x; 1.3465x over previous
import functools

import jax
import jax.numpy as jnp
from jax import lax
from jax.experimental import pallas as pl
from jax.experimental.pallas import tpu as pltpu
from jax.experimental.pallas import tpu_sc as plsc

N = 320000
D = 128
K = 1024
NC = 2
NS = 16
NW = NC * NS
LANES = 16
ROWS_W = N // NW
CH = 200
NCHUNK = ROWS_W // CH
AUXW = 2 * LANES

_mesh = plsc.VectorSubcoreMesh(core_axis_name="c", subcore_axis_name="s")


@functools.partial(
    pl.kernel,
    out_type=[
        jax.ShapeDtypeStruct((NC, K, D), jnp.float32),
        jax.ShapeDtypeStruct((NC, K, AUXW), jnp.float32),
    ],
    mesh=_mesh,
    compiler_params=pltpu.CompilerParams(use_tc_tiling_on_sc=False),
    scratch_types=[
        pltpu.VMEM((CH, D), jnp.float32),
        pltpu.VMEM((CH, D), jnp.float32),
        pltpu.VMEM((CH, AUXW), jnp.float32),
        pltpu.VMEM((CH, AUXW), jnp.float32),
        pltpu.VMEM((CH,), jnp.int32),
        pltpu.VMEM((CH,), jnp.int32),
        pltpu.VMEM_SHARED((K, D), jnp.float32),
        pltpu.VMEM_SHARED((K, AUXW), jnp.float32),
        pltpu.SemaphoreType.DMA,
        pltpu.SemaphoreType.DMA,
        pltpu.SemaphoreType.DMA,
        pltpu.SemaphoreType.DMA,
        pltpu.SemaphoreType.DMA,
        pltpu.SemaphoreType.DMA,
        pltpu.SemaphoreType.DMA,
        pltpu.SemaphoreType.DMA,
    ],
)
def _accum_kernel(var_hbm, lab_hbm, sums_out, aux_out,
                  x0, x1, z0, z1, i0, i1, ssums, saux,
                  inx0, inx1, inl0, inl1, scx0, scx1, scz0, scz1):
    cid = lax.axis_index("c")
    sid = lax.axis_index("s")
    wid = cid * NS + sid
    zeros16 = jnp.zeros((LANES,), jnp.float32)
    xb = (x0, x1)
    zb = (z0, z1)
    ib = (i0, i1)
    inx = (inx0, inx1)
    inl = (inl0, inl1)
    scx = (scx0, scx1)
    scz = (scz0, scz1)

    KSLICE = K // NS

    @plsc.parallel_loop(0, KSLICE, unroll=4)
    def _zrow(r):
        for j in range(D // LANES):
            x0[r, pl.ds(j * LANES, LANES)] = zeros16
        for j in range(AUXW // LANES):
            z0[r, pl.ds(j * LANES, LANES)] = zeros16

    pltpu.sync_copy(x0.at[pl.ds(0, KSLICE)],
                    ssums.at[pl.ds(sid * KSLICE, KSLICE)])
    pltpu.sync_copy(z0.at[pl.ds(0, KSLICE)],
                    saux.at[pl.ds(sid * KSLICE, KSLICE)])
    plsc.subcore_barrier()

    lanes = jnp.arange(LANES, dtype=jnp.int32)
    onehot = jnp.where(lanes == 0, 1.0, 0.0).astype(jnp.float32)

    @plsc.parallel_loop(0, CH, unroll=4)
    def _crow(r):
        z0[r, pl.ds(0, LANES)] = zeros16
        z0[r, pl.ds(LANES, LANES)] = onehot
        z1[r, pl.ds(0, LANES)] = zeros16
        z1[r, pl.ds(LANES, LANES)] = onehot

    row0 = wid * ROWS_W
    pltpu.async_copy(var_hbm.at[pl.ds(row0, CH)], x0, inx[0])
    pltpu.async_copy(lab_hbm.at[pl.ds(row0, CH)], i0, inl[0])

    def _pair(p, _):
        base_p = pl.multiple_of(row0 + p * 2 * CH, 8)
        for b in (0, 1):
            pltpu.make_async_copy(
                var_hbm.at[pl.ds(base_p + b * CH, CH)], xb[b],
                inx[b]).wait()
            pltpu.make_async_copy(
                lab_hbm.at[pl.ds(base_p + b * CH, CH)], ib[b],
                inl[b]).wait()

            xref = xb[b]
            zref = zb[b]

            @plsc.parallel_loop(0, CH, unroll=4)
            def _row(r):
                acc = zeros16
                for j in range(D // LANES):
                    v = xref[r, pl.ds(j * LANES, LANES)]
                    acc = acc + v * v
                zref[r, pl.ds(0, LANES)] = acc

            pltpu.async_copy(xb[b], ssums.at[ib[b]], scx[b], add=True)
            pltpu.async_copy(zb[b], saux.at[ib[b]], scz[b], add=True)

            ob = 1 - b
            nbase = pl.multiple_of(base_p + (b + 1) * CH, 8)

            def _prefetch(first):
                if not first:
                    pltpu.make_async_copy(
                        xb[ob], ssums.at[ib[ob]], scx[ob]).wait()
                    pltpu.make_async_copy(
                        zb[ob], saux.at[ib[ob]], scz[ob]).wait()
                pltpu.async_copy(var_hbm.at[pl.ds(nbase, CH)],
                                 xb[ob], inx[ob])
                pltpu.async_copy(lab_hbm.at[pl.ds(nbase, CH)],
                                 ib[ob], inl[ob])

            if b == 0:
                @pl.when(p > 0)
                def _():
                    _prefetch(False)

                @pl.when(p == 0)
                def _():
                    _prefetch(True)
            else:
                @pl.when(p + 1 < NCHUNK // 2)
                def _():
                    _prefetch(False)
        return 0
    lax.fori_loop(0, NCHUNK // 2, _pair, 0)

    for b in (0, 1):
        pltpu.make_async_copy(xb[b], ssums.at[ib[b]], scx[b]).wait()
        pltpu.make_async_copy(zb[b], saux.at[ib[b]], scz[b]).wait()

    plsc.subcore_barrier()

    pltpu.sync_copy(ssums.at[pl.ds(sid * KSLICE, KSLICE)],
                    sums_out.at[cid, pl.ds(sid * KSLICE, KSLICE)])
    pltpu.sync_copy(saux.at[pl.ds(sid * KSLICE, KSLICE)],
                    aux_out.at[cid, pl.ds(sid * KSLICE, KSLICE)])


SCH = K // NS


@functools.partial(
    pl.kernel,
    out_type=jax.ShapeDtypeStruct((LANES,), jnp.float32),
    mesh=_mesh,
    compiler_params=pltpu.CompilerParams(use_tc_tiling_on_sc=False),
    scratch_types=[
        pltpu.VMEM((SCH, D), jnp.float32),
        pltpu.VMEM((SCH, D), jnp.float32),
        pltpu.VMEM((SCH, AUXW), jnp.float32),
        pltpu.VMEM((SCH, AUXW), jnp.float32),
        pltpu.VMEM((D,), jnp.float32),
        pltpu.VMEM((NS, D), jnp.float32),
        pltpu.VMEM((LANES,), jnp.float32),
        pltpu.VMEM_SHARED((NS, D), jnp.float32),
    ],
)
def _loss_kernel(sums_p, aux_p, out_hbm, s0, s1, a0, a1,
                 stg, fold, outbuf, sstage):
    cid = lax.axis_index("c")
    sid = lax.axis_index("s")

    @pl.when(cid == 0)
    def _():
        lanes = jnp.arange(LANES, dtype=jnp.int32)
        zeros16 = jnp.zeros((LANES,), jnp.float32)
        off = sid * SCH
        pltpu.sync_copy(sums_p.at[0, pl.ds(off, SCH)], s0)
        pltpu.sync_copy(sums_p.at[1, pl.ds(off, SCH)], s1)
        pltpu.sync_copy(aux_p.at[0, pl.ds(off, SCH)], a0)
        pltpu.sync_copy(aux_p.at[1, pl.ds(off, SCH)], a1)

        def _seg(r, carry):
            loss_v, nuniq_s = carry
            sacc = zeros16
            for j in range(D // LANES):
                v = (s0[r, pl.ds(j * LANES, LANES)] +
                     s1[r, pl.ds(j * LANES, LANES)])
                sacc = sacc + v * v
            qv = a0[r, pl.ds(0, LANES)] + a1[r, pl.ds(0, LANES)]
            cv = (a0[r, pl.ds(LANES, LANES)] +
                  a1[r, pl.ds(LANES, LANES)])
            c = cv[0]

            segid = off + r
            present = c > 0.0
            valid = present & (segid != 0)
            safec = jnp.where(present, c, 1.0)
            contrib = jnp.where(
                valid, (qv - sacc / safec) / (safec * float(D)),
                zeros16)
            nuniq_s = nuniq_s + jnp.where(present, 1.0, 0.0)
            return loss_v + contrib, nuniq_s

        loss_v, nuniq = lax.fori_loop(0, SCH, _seg, (zeros16, 0.0))

        stg[pl.ds(0, LANES)] = loss_v
        stg[pl.ds(LANES, LANES)] = jnp.where(lanes == 0, nuniq, 0.0)
        for j in range(2, D // LANES):
            stg[pl.ds(j * LANES, LANES)] = zeros16
        pltpu.sync_copy(stg, sstage.at[sid])
        plsc.subcore_barrier()

        @pl.when(sid == 0)
        def _():
            pltpu.sync_copy(sstage, fold)
            lv = zeros16
            nv = zeros16
            for t in range(NS):
                lv = lv + fold[t, pl.ds(0, LANES)]
                nv = nv + fold[t, pl.ds(LANES, LANES)]
            loss = lv[0]
            for i in range(1, LANES):
                loss = loss + lv[i]
            nuniq_tot = nv[0]
            loss = jnp.where(nuniq_tot == 1.0, 0.0, loss)
            outbuf[pl.ds(0, LANES)] = jnp.where(lanes == 0, loss, 0.0)
            pltpu.sync_copy(outbuf, out_hbm)


def kernel(variances, ins_labels):
    sums_p, aux_p = _accum_kernel(variances, ins_labels.astype(jnp.int32))
    out = _loss_kernel(sums_p, aux_p)
    return out[0]
